# baseline (speedup 1.0000x reference)
import functools

import jax
import jax.numpy as jnp
from jax import lax
from jax.experimental import pallas as pl
from jax.experimental.pallas import tpu as pltpu
from jax.experimental.pallas import tpu_sc as plsc

NC = 2
NS = 16
L = 16
NW = NC * NS

CHUNK = 128


@functools.cache
def _build(M, D, B):
    assert B % L == 0, B
    assert M % 8 == 0, M
    MW = ((M + NW - 1) // NW + 7) // 8 * 8
    TAIL = M - (NW - 1) * MW
    assert 0 < TAIL <= MW, (M, MW, TAIL)
    PW = ((MW + L - 1) // L) * L
    NB = B // L

    mesh = plsc.VectorSubcoreMesh(
        core_axis_name="c", subcore_axis_name="s",
        num_cores=NC, num_subcores=NS)

    @functools.partial(
        pl.kernel,
        out_type=(
            jax.ShapeDtypeStruct((B, D), jnp.float32),
            jax.ShapeDtypeStruct((M, D), jnp.float32),
        ),
        mesh=mesh,
        compiler_params=pltpu.CompilerParams(needs_layout_passes=False, use_tc_tiling_on_sc=False),
        scratch_types=[
            pltpu.VMEM((B,), jnp.int32),
            pltpu.VMEM((PW,), jnp.int32),
            pltpu.VMEM((B + L,), jnp.int32),
            pltpu.VMEM((CHUNK,), jnp.int32),
            pltpu.VMEM((CHUNK,), jnp.int32),
            pltpu.VMEM((CHUNK,), jnp.int32),
            pltpu.VMEM((CHUNK, D), jnp.float32),
            pltpu.SemaphoreType.DMA,
            pltpu.SemaphoreType.DMA,
            pltpu.SemaphoreType.DMA,
            pltpu.SemaphoreType.DMA,
        ],
    )
    def k(mem_h, val_h, idx_h, out_h, newmem_h,
          idx_v, pos, plist, ibuf, kbuf, wbuf, rows,
          sem_c, sem_g, sem_s, sem_o):
        wid = lax.axis_index("s") * NC + lax.axis_index("c")
        lo = wid * MW
        hi = jnp.minimum(lo + MW, M)
        is_tail = wid == NW - 1
        iota = lax.iota(jnp.int32, L)

        pltpu.sync_copy(idx_h, idx_v)

        @pl.when(~is_tail)
        def _copy_full():
            pltpu.async_copy(
                mem_h.at[pl.ds(lo, MW), :], newmem_h.at[pl.ds(lo, MW), :],
                sem_c)

        @pl.when(is_tail)
        def _copy_tail():
            pltpu.async_copy(
                mem_h.at[pl.ds(lo, TAIL), :], newmem_h.at[pl.ds(lo, TAIL), :],
                sem_c)

        def scan_body(kk, c2):
            v = idx_v[pl.ds(kk * L, L)]
            i_vec = kk * L + iota
            inm = (v >= lo) & (v < hi)
            offs = c2 + lax.cumsum(inm.astype(jnp.int32), axis=0) - 1
            plsc.store_scatter(plist, [jnp.maximum(offs, 0)], i_vec, mask=inm)
            loc = jnp.minimum(jnp.maximum(v - lo, 0), MW - 1)
            plsc.store_scatter(pos, [loc], i_vec, mask=inm)
            p = plsc.load_gather(pos, [loc], mask=inm)
            lost = inm & (p != i_vec)
            nlost = jnp.sum(lost.astype(jnp.int32))

            @pl.when(nlost > 0)
            def _fix():
                for j in range(L):
                    plsc.store_scatter(pos, [loc], i_vec,
                                       mask=inm & (iota == j))

            return c2 + jnp.sum(inm.astype(jnp.int32))

        c2 = lax.fori_loop(0, NB, scan_body, jnp.int32(0))

        @pl.when(~is_tail)
        def _wait_full():
            pltpu.make_async_copy(
                mem_h.at[pl.ds(lo, MW), :], newmem_h.at[pl.ds(lo, MW), :],
                sem_c).wait()

        @pl.when(is_tail)
        def _wait_tail():
            pltpu.make_async_copy(
                mem_h.at[pl.ds(lo, TAIL), :], newmem_h.at[pl.ds(lo, TAIL), :],
                sem_c).wait()

        nch = (c2 + (CHUNK - 1)) // CHUNK

        def chunk_body(c, carry):
            last = c2 - 1
            for t in range(CHUNK // L):
                pj = jnp.minimum(c * CHUNK + t * L + iota, last)
                pv = plsc.load_gather(plist, [pj])
                kv = plsc.load_gather(idx_v, [pv])
                wv = plsc.load_gather(
                    pos, [jnp.minimum(jnp.maximum(kv - lo, 0), MW - 1)])
                ibuf[pl.ds(t * L, L)] = pv
                kbuf[pl.ds(t * L, L)] = kv
                wbuf[pl.ds(t * L, L)] = wv
            pltpu.async_copy(val_h.at[wbuf], rows, sem_g).wait()
            s1 = pltpu.async_copy(rows, newmem_h.at[kbuf], sem_s)
            s2 = pltpu.async_copy(rows, out_h.at[ibuf], sem_o)
            s1.wait()
            s2.wait()
            return carry

        lax.fori_loop(0, nch, chunk_body, jnp.int32(0))

    return k


def kernel(mem, val, idx):
    M, D = mem.shape
    B = idx.shape[0]
    out, new_mem = _build(M, D, B)(
        mem.astype(jnp.float32), val.astype(jnp.float32),
        idx.astype(jnp.int32))
    return out, new_mem


# speedup vs baseline: 6.2257x; 6.2257x over previous
import functools

import jax
import jax.numpy as jnp
from jax import lax
from jax.experimental import pallas as pl
from jax.experimental.pallas import tpu as pltpu
from jax.experimental.pallas import tpu_sc as plsc

NC = 2
NS = 16
L = 16
NW = NC * NS

CHUNK = 128
CR = 256


@functools.cache
def _build(M, D, B):
    assert B % L == 0, B
    assert M % 8 == 0, M
    MW = ((M + NW - 1) // NW + 7) // 8 * 8
    TAIL = M - (NW - 1) * MW
    assert 0 < TAIL <= MW, (M, MW, TAIL)
    PW = ((MW + L - 1) // L) * L
    NB = B // L

    NFC = MW // CR // 2 * 2
    NFT = TAIL // CR // 2 * 2
    assert NFC >= 2 and NFT >= 2, (NFC, NFT)
    SPP = -(-NB // (min(NFC, NFT) // 2))

    mesh = plsc.VectorSubcoreMesh(
        core_axis_name="c", subcore_axis_name="s",
        num_cores=NC, num_subcores=NS)

    @functools.partial(
        pl.kernel,
        out_type=(
            jax.ShapeDtypeStruct((B, D), jnp.float32),
            jax.ShapeDtypeStruct((M, D), jnp.float32),
        ),
        mesh=mesh,
        compiler_params=pltpu.CompilerParams(
            needs_layout_passes=False, use_tc_tiling_on_sc=False),
        scratch_types=[
            pltpu.VMEM((B,), jnp.int32),
            pltpu.VMEM((PW,), jnp.int32),
            pltpu.VMEM((B + L,), jnp.int32),
            pltpu.VMEM((CHUNK,), jnp.int32),
            pltpu.VMEM((CHUNK,), jnp.int32),
            pltpu.VMEM((CHUNK,), jnp.int32),
            pltpu.VMEM((CHUNK, D), jnp.float32),
            pltpu.VMEM((CR, D), jnp.float32),
            pltpu.VMEM((CR, D), jnp.float32),
            pltpu.SemaphoreType.DMA,
            pltpu.SemaphoreType.DMA,
            pltpu.SemaphoreType.DMA,
            pltpu.SemaphoreType.DMA,
            pltpu.SemaphoreType.DMA,
            pltpu.SemaphoreType.DMA,
            pltpu.SemaphoreType.DMA,
        ],
    )
    def k(mem_h, val_h, idx_h, out_h, newmem_h,
          idx_v, pos, plist, ibuf, kbuf, wbuf, rows, cb0, cb1,
          si0, si1, so0, so1, sem_g, sem_s, sem_o):
        wid = lax.axis_index("s") * NC + lax.axis_index("c")
        lo = wid * MW
        hi = jnp.minimum(lo + MW, M)
        is_tail = wid == NW - 1
        nfc = jnp.where(is_tail, NFT, NFC)
        npair = nfc // 2
        iota = lax.iota(jnp.int32, L)

        pltpu.sync_copy(idx_h, idx_v)

        def cin(c, buf, sem):
            return pltpu.async_copy(
                mem_h.at[pl.ds(lo + c * CR, CR), :], buf, sem)

        def cin_wait(buf, sem):
            pltpu.make_async_copy(
                mem_h.at[pl.ds(lo, CR), :], buf, sem).wait()

        def cout(c, buf, sem):
            return pltpu.async_copy(
                buf, newmem_h.at[pl.ds(lo + c * CR, CR), :], sem)

        def cout_wait(buf, sem):
            pltpu.make_async_copy(
                buf, newmem_h.at[pl.ds(lo, CR), :], sem).wait()

        def scan_step(kk, c2):
            koff = jnp.minimum(kk, NB - 1) * L
            v = idx_v[pl.ds(koff, L)]
            i_vec = koff + iota
            inm = (v >= lo) & (v < hi) & (kk < NB)
            offs = c2 + lax.cumsum(inm.astype(jnp.int32), axis=0) - 1
            plsc.store_scatter(plist, [jnp.maximum(offs, 0)], i_vec, mask=inm)
            loc = jnp.minimum(jnp.maximum(v - lo, 0), MW - 1)
            plsc.store_scatter(pos, [loc], i_vec, mask=inm)
            p = plsc.load_gather(pos, [loc], mask=inm)
            lost = inm & (p != i_vec)
            nlost = jnp.sum(lost.astype(jnp.int32))

            @pl.when(nlost > 0)
            def _fix():
                for j in range(L):
                    plsc.store_scatter(pos, [loc], i_vec,
                                       mask=inm & (iota == j))

            return c2 + jnp.sum(inm.astype(jnp.int32))

        cin(0, cb0, si0)
        cin(1, cb1, si1)

        def pair_body(j, c2):
            for t in range(SPP):
                c2 = scan_step(j * SPP + t, c2)
            c = 2 * j
            cin_wait(cb0, si0)
            cout(c, cb0, so0)
            cin_wait(cb1, si1)
            cout(c + 1, cb1, so1)

            @pl.when(j + 1 < npair)
            def _prefetch():
                cout_wait(cb0, so0)
                cin(c + 2, cb0, si0)
                cout_wait(cb1, so1)
                cin(c + 3, cb1, si1)

            return c2

        c2 = lax.fori_loop(0, npair, pair_body, jnp.int32(0))
        cout_wait(cb0, so0)
        cout_wait(cb1, so1)

        def drain_rest(nfull, total_rows):
            off = nfull * CR
            while off < total_rows:
                sz = min(total_rows - off, CR)
                pltpu.sync_copy(mem_h.at[pl.ds(lo + off, sz), :],
                                cb0.at[pl.ds(0, sz), :])
                pltpu.sync_copy(cb0.at[pl.ds(0, sz), :],
                                newmem_h.at[pl.ds(lo + off, sz), :])
                off += sz

        if NFC * CR < MW:
            @pl.when(~is_tail)
            def _rem_full():
                drain_rest(NFC, MW)
        if NFT * CR < TAIL:
            @pl.when(is_tail)
            def _rem_tail():
                drain_rest(NFT, TAIL)

        nch = (c2 + (CHUNK - 1)) // CHUNK

        def chunk_body(c, carry):
            last = c2 - 1
            for t in range(CHUNK // L):
                pj = jnp.minimum(c * CHUNK + t * L + iota, last)
                pv = plsc.load_gather(plist, [pj])
                kv = plsc.load_gather(idx_v, [pv])
                wv = plsc.load_gather(
                    pos, [jnp.minimum(jnp.maximum(kv - lo, 0), MW - 1)])
                ibuf[pl.ds(t * L, L)] = pv
                kbuf[pl.ds(t * L, L)] = kv
                wbuf[pl.ds(t * L, L)] = wv
            pltpu.async_copy(val_h.at[wbuf], rows, sem_g).wait()
            s1 = pltpu.async_copy(rows, newmem_h.at[kbuf], sem_s)
            s2 = pltpu.async_copy(rows, out_h.at[ibuf], sem_o)
            s1.wait()
            s2.wait()
            return carry

        lax.fori_loop(0, nch, chunk_body, jnp.int32(0))

    return k


def kernel(mem, val, idx):
    M, D = mem.shape
    B = idx.shape[0]
    out, new_mem = _build(M, D, B)(
        mem.astype(jnp.float32), val.astype(jnp.float32),
        idx.astype(jnp.int32))
    return out, new_mem
